# baseline (speedup 1.0000x reference)
import functools

import jax
import jax.numpy as jnp
from jax import lax
from jax.experimental import pallas as pl
from jax.experimental.pallas import tpu as pltpu
from jax.experimental.pallas import tpu_sc as plsc

N = 10000
D = 128
NPAD = 10240
NC, NS = 2, 16
NW = NC * NS
CHUNK = 128
ROWS_PER_TILE = NPAD // NS
CW = 16

_MESH = plsc.VectorSubcoreMesh(
    core_axis_name="c", subcore_axis_name="s", num_cores=NC, num_subcores=NS
)


def _count_body(nchunk, dst_hbm, out_hbm, dst_v, ones_v, zero_v, acc_sh, sem):
    c = lax.axis_index("c")
    s = lax.axis_index("s")
    wid = c * NS + s

    def fill(i, _):
        ones_v[i, :] = jnp.ones((16,), jnp.float32)
        zero_v[i, :] = jnp.zeros((16,), jnp.float32)
        return 0

    lax.fori_loop(0, CHUNK, fill, 0)
    for k in range(ROWS_PER_TILE // CHUNK):
        pltpu.sync_copy(zero_v, acc_sh.at[pl.ds(s * ROWS_PER_TILE + k * CHUNK, CHUNK)])
    pltpu.sync_copy(dst_hbm.at[wid], dst_v)
    plsc.subcore_barrier()

    def body(j, _):
        pltpu.async_copy(ones_v, acc_sh.at[dst_v.at[j]], sem, add=True).wait()
        return 0

    lax.fori_loop(0, nchunk, body, 0)
    plsc.subcore_barrier()
    pltpu.sync_copy(
        acc_sh.at[pl.ds(s * ROWS_PER_TILE, ROWS_PER_TILE)],
        out_hbm.at[c, pl.ds(s * ROWS_PER_TILE, ROWS_PER_TILE)],
    )


GRP = 8


def _scatter_body(nchunk, h_hbm, src_hbm, dst_hbm, out_hbm,
                  src_v, dst_v, rowbuf, acc_sh, gsem, ssem, isem):
    ngroups = nchunk // GRP
    c = lax.axis_index("c")
    s = lax.axis_index("s")
    wid = c * NS + s

    def zrow(i, _):
        for jj in range(D // 16):
            rowbuf[0, i, pl.ds(jj * 16, 16)] = jnp.zeros((16,), jnp.float32)
        return 0

    lax.fori_loop(0, CHUNK, zrow, 0)
    for k in range(ROWS_PER_TILE // CHUNK):
        pltpu.sync_copy(rowbuf.at[0], acc_sh.at[pl.ds(s * ROWS_PER_TILE + k * CHUNK, CHUNK)])
    plsc.subcore_barrier()

    def _drain_scat(b):
        pltpu.make_async_copy(h_hbm.at[pl.ds(0, CHUNK)], rowbuf.at[b], ssem).wait()

    def _drain_idx(gb):
        pltpu.make_async_copy(src_hbm.at[wid, pl.ds(0, GRP)], src_v.at[gb], isem).wait()
        pltpu.make_async_copy(src_hbm.at[wid, pl.ds(0, GRP)], dst_v.at[gb], isem).wait()

    def _issue_idx(g, gb):
        pltpu.async_copy(src_hbm.at[wid, pl.ds(GRP * g, GRP)], src_v.at[gb], isem)
        pltpu.async_copy(dst_hbm.at[wid, pl.ds(GRP * g, GRP)], dst_v.at[gb], isem)

    def _step(gb, t, drain):
        b = t % 2
        if drain:
            _drain_scat(b)
        pltpu.async_copy(h_hbm.at[src_v.at[gb, t]], rowbuf.at[b], gsem).wait()
        pltpu.async_copy(rowbuf.at[b], acc_sh.at[dst_v.at[gb, t]], ssem, add=True)

    _issue_idx(0, 0)
    _drain_idx(0)
    _issue_idx(1, 1)
    for t in range(GRP):
        _step(0, t, t >= 2)

    def body(g, _):
        gb = g % 2
        _drain_idx(gb)
        _step(gb, 0, True)
        _step(gb, 1, True)

        @pl.when(g < ngroups - 1)
        def _():
            _issue_idx(g + 1, (g + 1) % 2)

        for t in range(2, GRP):
            _step(gb, t, True)
        return 0

    lax.fori_loop(1, ngroups, body, 0)
    for b in range(2):
        _drain_scat(b)
    plsc.subcore_barrier()
    pltpu.sync_copy(
        acc_sh.at[pl.ds(s * ROWS_PER_TILE, ROWS_PER_TILE)],
        out_hbm.at[c, pl.ds(s * ROWS_PER_TILE, ROWS_PER_TILE)],
    )


def _make_count(nchunk):
    return pl.kernel(
        functools.partial(_count_body, nchunk),
        out_type=jax.ShapeDtypeStruct((NC, NPAD, CW), jnp.float32),
        mesh=_MESH,
        scratch_types=[
            pltpu.VMEM((nchunk, CHUNK), jnp.int32),
            pltpu.VMEM((CHUNK, CW), jnp.float32),
            pltpu.VMEM((CHUNK, CW), jnp.float32),
            pltpu.VMEM_SHARED((NPAD, CW), jnp.float32),
            pltpu.SemaphoreType.DMA,
        ],
    )


def _make_scatter(nchunk):
    return pl.kernel(
        functools.partial(_scatter_body, nchunk),
        out_type=jax.ShapeDtypeStruct((NC, NPAD, D), jnp.float32),
        mesh=_MESH,
        scratch_types=[
            pltpu.VMEM((2, GRP, CHUNK), jnp.int32),
            pltpu.VMEM((2, GRP, CHUNK), jnp.int32),
            pltpu.VMEM((2, CHUNK, D), jnp.float32),
            pltpu.VMEM_SHARED((NPAD, D), jnp.float32),
            pltpu.SemaphoreType.DMA,
            pltpu.SemaphoreType.DMA,
            pltpu.SemaphoreType.DMA,
        ],
    )


BM = 1024


def _tc1_body(cnt0_ref, cnt1_ref, x_ref, w_ref, h_ref, dinv_ref):
    deg = cnt0_ref[...] + cnt1_ref[...] + 1.0
    dinv = lax.rsqrt(deg)
    h = jnp.dot(x_ref[...], w_ref[...], preferred_element_type=jnp.float32)
    h_ref[...] = h * dinv
    dinv_ref[...] = dinv


def _tc2_body(a0_ref, a1_ref, hp_ref, dinv_ref, b_ref, w_ref, o_ref):
    dinv = dinv_ref[...]
    z = dinv * (a0_ref[...] + a1_ref[...] + hp_ref[...]) + b_ref[...]
    z = jnp.maximum(z, 0.0)
    o_ref[...] = jnp.dot(z, w_ref[...], preferred_element_type=jnp.float32) * dinv


def _tc3_body(a0_ref, a1_ref, hp_ref, dinv_ref, b_ref, x_ref, o_ref):
    h = dinv_ref[...] * (a0_ref[...] + a1_ref[...] + hp_ref[...]) + b_ref[...]
    o_ref[...] = 0.8 * h + 0.2 * x_ref[...]


def _row_spec(w):
    return pl.BlockSpec((BM, w), lambda i: (i, 0))


def _full_spec(r, w):
    return pl.BlockSpec((r, w), lambda i: (0, 0))


_tc1 = pl.pallas_call(
    _tc1_body,
    grid=(NPAD // BM,),
    in_specs=[_row_spec(1), _row_spec(1), _row_spec(D), _full_spec(D, D)],
    out_specs=[_row_spec(D), _row_spec(1)],
    out_shape=[
        jax.ShapeDtypeStruct((NPAD, D), jnp.float32),
        jax.ShapeDtypeStruct((NPAD, 1), jnp.float32),
    ],
)

_tc2 = pl.pallas_call(
    _tc2_body,
    grid=(NPAD // BM,),
    in_specs=[_row_spec(D), _row_spec(D), _row_spec(D), _row_spec(1),
              _full_spec(1, D), _full_spec(D, D)],
    out_specs=_row_spec(D),
    out_shape=jax.ShapeDtypeStruct((NPAD, D), jnp.float32),
)

_tc3 = pl.pallas_call(
    _tc3_body,
    grid=(NPAD // BM,),
    in_specs=[_row_spec(D), _row_spec(D), _row_spec(D), _row_spec(1),
              _full_spec(1, D), _row_spec(D)],
    out_specs=_row_spec(D),
    out_shape=jax.ShapeDtypeStruct((NPAD, D), jnp.float32),
)


def kernel(x, edge_index, layers, W1, b1, W2, b2):
    e = edge_index.shape[1]
    grain = NW * CHUNK * GRP
    epad = ((e + grain - 1) // grain) * grain
    nchunk = epad // (NW * CHUNK)

    ei = edge_index.astype(jnp.int32)
    pad = jnp.full((epad - e,), N, dtype=jnp.int32)
    src_t = jnp.concatenate([ei[0], pad]).reshape(NW, nchunk, CHUNK)
    dst_t = jnp.concatenate([ei[1], pad]).reshape(NW, nchunk, CHUNK)
    x_pad = jnp.pad(x, ((0, NPAD - N), (0, 0)))
    b1r = b1.reshape(1, D)
    b2r = b2.reshape(1, D)

    cnt = _make_count(nchunk)(dst_t)
    cnt0 = cnt[0, :, 0:1]
    cnt1 = cnt[1, :, 0:1]

    h1p, dinv = _tc1(cnt0, cnt1, x_pad, W1)

    scat = _make_scatter(nchunk)
    acc1 = scat(h1p, src_t, dst_t)
    h2p = _tc2(acc1[0], acc1[1], h1p, dinv, b1r, W2)

    acc2 = scat(h2p, src_t, dst_t)
    out = _tc3(acc2[0], acc2[1], h2p, dinv, b2r, x_pad)
    return out[:N]


# speedup vs baseline: 1.3757x; 1.3757x over previous
import functools

import jax
import jax.numpy as jnp
from jax import lax
from jax.experimental import pallas as pl
from jax.experimental.pallas import tpu as pltpu
from jax.experimental.pallas import tpu_sc as plsc

N = 10000
D = 128
NPAD = 10240
NC, NS = 2, 16
NW = NC * NS
CHUNK = 128
ROWS_PER_TILE = NPAD // NS
CW = 16

_MESH = plsc.VectorSubcoreMesh(
    core_axis_name="c", subcore_axis_name="s", num_cores=NC, num_subcores=NS
)


def _count_body(nchunk, dst_hbm, out_hbm, dst_v, ones_v, zero_v, acc_sh, sem):
    c = lax.axis_index("c")
    s = lax.axis_index("s")
    wid = c * NS + s

    def fill(i, _):
        ones_v[i, :] = jnp.ones((16,), jnp.float32)
        zero_v[i, :] = jnp.zeros((16,), jnp.float32)
        return 0

    lax.fori_loop(0, CHUNK, fill, 0)
    for k in range(ROWS_PER_TILE // CHUNK):
        pltpu.sync_copy(zero_v, acc_sh.at[pl.ds(s * ROWS_PER_TILE + k * CHUNK, CHUNK)])
    pltpu.sync_copy(dst_hbm.at[wid], dst_v)
    plsc.subcore_barrier()

    def body(j, _):
        pltpu.async_copy(ones_v, acc_sh.at[dst_v.at[j]], sem, add=True).wait()
        return 0

    lax.fori_loop(0, nchunk, body, 0)
    plsc.subcore_barrier()
    pltpu.sync_copy(
        acc_sh.at[pl.ds(s * ROWS_PER_TILE, ROWS_PER_TILE)],
        out_hbm.at[c, pl.ds(s * ROWS_PER_TILE, ROWS_PER_TILE)],
    )


def _scatter_body(nchunk, h_hbm, src_hbm, dst_hbm, out_hbm,
                  src_v, dst_v, rowbuf, acc_sh, gsem, ssem):
    c = lax.axis_index("c")
    s = lax.axis_index("s")
    wid = c * NS + s

    def zrow(i, _):
        for jj in range(D // 16):
            rowbuf[i, pl.ds(jj * 16, 16)] = jnp.zeros((16,), jnp.float32)
        return 0

    lax.fori_loop(0, CHUNK, zrow, 0)
    for k in range(ROWS_PER_TILE // CHUNK):
        pltpu.sync_copy(rowbuf, acc_sh.at[pl.ds(s * ROWS_PER_TILE + k * CHUNK, CHUNK)])
    pltpu.sync_copy(src_hbm.at[wid], src_v)
    pltpu.sync_copy(dst_hbm.at[wid], dst_v)
    plsc.subcore_barrier()

    def body(j, _):
        pltpu.async_copy(h_hbm.at[src_v.at[j]], rowbuf, gsem).wait()
        pltpu.async_copy(rowbuf, acc_sh.at[dst_v.at[j]], ssem, add=True).wait()
        return 0

    lax.fori_loop(0, nchunk, body, 0)
    plsc.subcore_barrier()
    pltpu.sync_copy(
        acc_sh.at[pl.ds(s * ROWS_PER_TILE, ROWS_PER_TILE)],
        out_hbm.at[c, pl.ds(s * ROWS_PER_TILE, ROWS_PER_TILE)],
    )


def _make_count(nchunk):
    return pl.kernel(
        functools.partial(_count_body, nchunk),
        out_type=jax.ShapeDtypeStruct((NC, NPAD, CW), jnp.float32),
        mesh=_MESH,
        scratch_types=[
            pltpu.VMEM((nchunk, CHUNK), jnp.int32),
            pltpu.VMEM((CHUNK, CW), jnp.float32),
            pltpu.VMEM((CHUNK, CW), jnp.float32),
            pltpu.VMEM_SHARED((NPAD, CW), jnp.float32),
            pltpu.SemaphoreType.DMA,
        ],
    )


def _make_scatter(nchunk):
    return pl.kernel(
        functools.partial(_scatter_body, nchunk),
        out_type=jax.ShapeDtypeStruct((NC, NPAD, D), jnp.float32),
        mesh=_MESH,
        scratch_types=[
            pltpu.VMEM((nchunk, CHUNK), jnp.int32),
            pltpu.VMEM((nchunk, CHUNK), jnp.int32),
            pltpu.VMEM((CHUNK, D), jnp.float32),
            pltpu.VMEM_SHARED((NPAD, D), jnp.float32),
            pltpu.SemaphoreType.DMA,
            pltpu.SemaphoreType.DMA,
        ],
    )


BM = 1024


def _tc1_body(cnt0_ref, cnt1_ref, x_ref, w_ref, h_ref, dinv_ref):
    deg = cnt0_ref[...] + cnt1_ref[...] + 1.0
    dinv = lax.rsqrt(deg)
    h = jnp.dot(x_ref[...], w_ref[...], preferred_element_type=jnp.float32)
    h_ref[...] = h * dinv
    dinv_ref[...] = dinv


def _tc2_body(a0_ref, a1_ref, hp_ref, dinv_ref, b_ref, w_ref, o_ref):
    dinv = dinv_ref[...]
    z = dinv * (a0_ref[...] + a1_ref[...] + hp_ref[...]) + b_ref[...]
    z = jnp.maximum(z, 0.0)
    o_ref[...] = jnp.dot(z, w_ref[...], preferred_element_type=jnp.float32) * dinv


def _tc3_body(a0_ref, a1_ref, hp_ref, dinv_ref, b_ref, x_ref, o_ref):
    h = dinv_ref[...] * (a0_ref[...] + a1_ref[...] + hp_ref[...]) + b_ref[...]
    o_ref[...] = 0.8 * h + 0.2 * x_ref[...]


def _row_spec(w):
    return pl.BlockSpec((BM, w), lambda i: (i, 0))


def _full_spec(r, w):
    return pl.BlockSpec((r, w), lambda i: (0, 0))


_tc1 = pl.pallas_call(
    _tc1_body,
    grid=(NPAD // BM,),
    in_specs=[_row_spec(1), _row_spec(1), _row_spec(D), _full_spec(D, D)],
    out_specs=[_row_spec(D), _row_spec(1)],
    out_shape=[
        jax.ShapeDtypeStruct((NPAD, D), jnp.float32),
        jax.ShapeDtypeStruct((NPAD, 1), jnp.float32),
    ],
)

_tc2 = pl.pallas_call(
    _tc2_body,
    grid=(NPAD // BM,),
    in_specs=[_row_spec(D), _row_spec(D), _row_spec(D), _row_spec(1),
              _full_spec(1, D), _full_spec(D, D)],
    out_specs=_row_spec(D),
    out_shape=jax.ShapeDtypeStruct((NPAD, D), jnp.float32),
)

_tc3 = pl.pallas_call(
    _tc3_body,
    grid=(NPAD // BM,),
    in_specs=[_row_spec(D), _row_spec(D), _row_spec(D), _row_spec(1),
              _full_spec(1, D), _row_spec(D)],
    out_specs=_row_spec(D),
    out_shape=jax.ShapeDtypeStruct((NPAD, D), jnp.float32),
)


def kernel(x, edge_index, layers, W1, b1, W2, b2):
    e = edge_index.shape[1]
    grain = NW * CHUNK
    epad = ((e + grain - 1) // grain) * grain
    nchunk = epad // grain

    ei = edge_index.astype(jnp.int32)
    pad = jnp.full((epad - e,), N, dtype=jnp.int32)
    src_t = jnp.concatenate([ei[0], pad]).reshape(NW, nchunk, CHUNK)
    dst_t = jnp.concatenate([ei[1], pad]).reshape(NW, nchunk, CHUNK)
    x_pad = jnp.pad(x, ((0, NPAD - N), (0, 0)))
    b1r = b1.reshape(1, D)
    b2r = b2.reshape(1, D)

    cnt = _make_count(nchunk)(dst_t)
    cnt0 = cnt[0, :, 0:1]
    cnt1 = cnt[1, :, 0:1]

    h1p, dinv = _tc1(cnt0, cnt1, x_pad, W1)

    scat = _make_scatter(nchunk)
    acc1 = scat(h1p, src_t, dst_t)
    h2p = _tc2(acc1[0], acc1[1], h1p, dinv, b1r, W2)

    acc2 = scat(h2p, src_t, dst_t)
    out = _tc3(acc2[0], acc2[1], h2p, dinv, b2r, x_pad)
    return out[:N]
